# Initial kernel scaffold; baseline (speedup 1.0000x reference)
import jax
import jax.numpy as jnp
from jax.experimental import pallas as pl


def kernel(x, centroids, W1, b1, W2, b2):
    raise NotImplementedError("write your pallas kernel here")


# speedup vs baseline: 3.9738x; 3.9738x over previous
import functools

import jax
import jax.numpy as jnp
from jax import lax
from jax.experimental import pallas as pl
from jax.experimental.pallas import tpu as pltpu

_TOPK = 2
_KBLK = 512


def _router_gates(u, centroids):
    scores = u @ centroids.T
    gates = jax.nn.sigmoid(scores)
    topk_vals, topk_idx = jax.lax.top_k(gates, _TOPK)
    n = gates.shape[0]
    mask = jnp.zeros_like(gates).at[jnp.arange(n)[:, None], topk_idx].set(topk_vals)
    return mask / jnp.sum(mask, axis=-1, keepdims=True)


def _gelu_exact(h):
    return 0.5 * h * (1.0 + lax.erf(h * (2.0 ** -0.5)))


def _ffn_body(gates_ref, x_ref, w1_ref, b1_ref, w2_ref, b2_ref, out_ref):
    e = pl.program_id(0)
    k = pl.program_id(1)

    @pl.when((e == 0) & (k == 0))
    def _init():
        out_ref[...] = jnp.zeros_like(out_ref)

    xb = x_ref[...].astype(jnp.bfloat16)
    w1 = w1_ref[0].astype(jnp.bfloat16)
    h = lax.dot_general(xb, w1, (((1,), (0,)), ((), ())),
                        preferred_element_type=jnp.float32)
    h = h + b1_ref[0, 0, pl.ds(k * _KBLK, _KBLK)][None, :]
    h = _gelu_exact(h)
    w2 = w2_ref[0].astype(jnp.bfloat16)
    y = lax.dot_general(h.astype(jnp.bfloat16), w2, (((1,), (0,)), ((), ())),
                        preferred_element_type=jnp.float32)
    g = gates_ref[0, 0, :][:, None]
    out_ref[...] += g * y

    @pl.when(k == 0)
    def _bias():
        out_ref[...] += g * b2_ref[0, 0][None, :]


def _ffn(gates, u, W1, b1, W2, b2):
    n, d = u.shape
    e_num, _, dff = W1.shape
    nk = dff // _KBLK
    grid = (e_num, nk)
    return pl.pallas_call(
        _ffn_body,
        grid=grid,
        in_specs=[
            pl.BlockSpec((1, 1, n), lambda e, k: (e, 0, 0)),
            pl.BlockSpec((n, d), lambda e, k: (0, 0)),
            pl.BlockSpec((1, d, _KBLK), lambda e, k: (e, 0, k)),
            pl.BlockSpec((1, 1, dff), lambda e, k: (e, 0, 0)),
            pl.BlockSpec((1, _KBLK, d), lambda e, k: (e, k, 0)),
            pl.BlockSpec((1, 1, d), lambda e, k: (e, 0, 0)),
        ],
        out_specs=pl.BlockSpec((n, d), lambda e, k: (0, 0)),
        out_shape=jax.ShapeDtypeStruct((n, d), jnp.float32),
        compiler_params=pltpu.CompilerParams(
            dimension_semantics=("arbitrary", "arbitrary"),
        ),
    )(gates.T.reshape(e_num, 1, n), u, W1, b1.reshape(e_num, 1, dff),
      W2, b2.reshape(e_num, 1, d))


def kernel(x, centroids, W1, b1, W2, b2):
    b, s, d = x.shape
    u = x.reshape(b * s, d)
    gates = _router_gates(u, centroids)
    out = _ffn(gates, u, W1, b1, W2, b2)
    return out.reshape(b, s, d)


# speedup vs baseline: 5.0727x; 1.2765x over previous
import functools

import jax
import jax.numpy as jnp
from jax import lax
from jax.experimental import pallas as pl
from jax.experimental.pallas import tpu as pltpu
from jax.experimental.pallas import tpu_sc as plsc

_BLK = 512
_NBLK = 15
_RMAX = _NBLK * _BLK
_KBLK = 512
_NTILES = 32


def _gelu_exact(h):
    return 0.5 * h * (1.0 + lax.erf(h * (2.0 ** -0.5)))


def _make_dispatch(n, d):
    chunk = n // _NTILES
    mesh = plsc.VectorSubcoreMesh(core_axis_name="c", subcore_axis_name="s")

    @functools.partial(
        pl.kernel,
        mesh=mesh,
        out_type=jax.ShapeDtypeStruct((_RMAX, d), jnp.float32),
        scratch_types=[
            pltpu.VMEM((chunk,), jnp.int32),
            pltpu.VMEM((chunk,), jnp.int32),
            pltpu.VMEM((chunk, d), jnp.float32),
            pltpu.SemaphoreType.DMA,
            pltpu.SemaphoreType.DMA,
        ],
    )
    def dispatch(x_hbm, r0_hbm, r1_hbm, xs_hbm, i0_v, i1_v, x_v, s0, s1):
        wid = lax.axis_index("s") * 2 + lax.axis_index("c")
        base = wid * chunk
        pltpu.sync_copy(r0_hbm.at[pl.ds(base, chunk)], i0_v)
        pltpu.sync_copy(r1_hbm.at[pl.ds(base, chunk)], i1_v)
        pltpu.sync_copy(x_hbm.at[pl.ds(base, chunk)], x_v)
        c0 = pltpu.async_copy(x_v, xs_hbm.at[i0_v], s0)
        c1 = pltpu.async_copy(x_v, xs_hbm.at[i1_v], s1)
        c0.wait()
        c1.wait()

    return dispatch


def _make_combine(n, d):
    chunk = n // _NTILES
    mesh = plsc.VectorSubcoreMesh(core_axis_name="c", subcore_axis_name="s")

    @functools.partial(
        pl.kernel,
        mesh=mesh,
        out_type=jax.ShapeDtypeStruct((n, d), jnp.float32),
        scratch_types=[
            pltpu.VMEM((chunk,), jnp.int32),
            pltpu.VMEM((chunk,), jnp.int32),
            pltpu.VMEM((chunk, d), jnp.float32),
            pltpu.VMEM((chunk, d), jnp.float32),
            pltpu.SemaphoreType.DMA,
            pltpu.SemaphoreType.DMA,
        ],
    )
    def combine(ys_hbm, r0_hbm, r1_hbm, out_hbm, i0_v, i1_v, a_v, b_v, s0, s1):
        wid = lax.axis_index("s") * 2 + lax.axis_index("c")
        base = wid * chunk
        pltpu.sync_copy(r0_hbm.at[pl.ds(base, chunk)], i0_v)
        pltpu.sync_copy(r1_hbm.at[pl.ds(base, chunk)], i1_v)
        c0 = pltpu.async_copy(ys_hbm.at[i0_v], a_v, s0)
        c1 = pltpu.async_copy(ys_hbm.at[i1_v], b_v, s1)
        c0.wait()
        c1.wait()

        nvec = d // 16

        def tok_body(t, carry):
            def col_body(c, carry2):
                sl = pl.ds(c * 16, 16)
                a_v[t, sl] = a_v[t, sl] + b_v[t, sl]
                return carry2
            return lax.fori_loop(0, nvec, col_body, carry, unroll=8)

        lax.fori_loop(0, chunk, tok_body, 0)
        pltpu.sync_copy(a_v, out_hbm.at[pl.ds(base, chunk)])

    return combine


def _ffn_body(be_ref, nv_ref, xs_ref, w1_ref, b1_ref, w2_ref, b2_ref, ws_ref,
              out_ref):
    j = pl.program_id(0)

    @pl.when(j < nv_ref[0])
    def _compute():
        xb = xs_ref[...].astype(jnp.bfloat16)
        d = xs_ref.shape[1]
        acc = jnp.zeros((_BLK, d), jnp.float32)
        nk = w1_ref.shape[2] // _KBLK
        for k in range(nk):
            sl = slice(k * _KBLK, (k + 1) * _KBLK)
            w1k = w1_ref[0, :, sl].astype(jnp.bfloat16)
            h = lax.dot_general(xb, w1k, (((1,), (0,)), ((), ())),
                                preferred_element_type=jnp.float32)
            h = _gelu_exact(h + b1_ref[0, 0, sl][None, :])
            w2k = w2_ref[0, sl, :].astype(jnp.bfloat16)
            acc += lax.dot_general(h.astype(jnp.bfloat16), w2k,
                                   (((1,), (0,)), ((), ())),
                                   preferred_element_type=jnp.float32)
        y = acc + b2_ref[0, 0][None, :]
        out_ref[...] = y * ws_ref[0, 0][:, None]


def _grouped_ffn(block_expert, nvalid, xs, W1, b1, W2, b2, w_sorted):
    e_num, d, dff = W1.shape
    grid_spec = pltpu.PrefetchScalarGridSpec(
        num_scalar_prefetch=2,
        grid=(_NBLK,),
        in_specs=[
            pl.BlockSpec((_BLK, d),
                         lambda j, be, nv: (jnp.minimum(j, nv[0] - 1), 0)),
            pl.BlockSpec((1, d, dff), lambda j, be, nv: (be[j], 0, 0)),
            pl.BlockSpec((1, 1, dff), lambda j, be, nv: (be[j], 0, 0)),
            pl.BlockSpec((1, dff, d), lambda j, be, nv: (be[j], 0, 0)),
            pl.BlockSpec((1, 1, d), lambda j, be, nv: (be[j], 0, 0)),
            pl.BlockSpec((1, 1, _BLK), lambda j, be, nv: (j, 0, 0)),
        ],
        out_specs=pl.BlockSpec((_BLK, d), lambda j, be, nv: (j, 0)),
    )
    return pl.pallas_call(
        _ffn_body,
        grid_spec=grid_spec,
        out_shape=jax.ShapeDtypeStruct((_RMAX, d), jnp.float32),
        compiler_params=pltpu.CompilerParams(
            dimension_semantics=("arbitrary",),
        ),
    )(block_expert, nvalid, xs, W1, b1.reshape(e_num, 1, dff), W2,
      b2.reshape(e_num, 1, d), w_sorted.reshape(_NBLK, 1, _BLK))


def kernel(x, centroids, W1, b1, W2, b2):
    b, s, d = x.shape
    n = b * s
    e_num = centroids.shape[0]
    u = x.reshape(n, d)

    scores = u @ centroids.T
    gates = jax.nn.sigmoid(scores)
    topk_vals, topk_idx = jax.lax.top_k(gates, 2)
    wnorm = topk_vals / jnp.sum(topk_vals, axis=-1, keepdims=True)

    flat_e = topk_idx.reshape(-1)
    onehot = (flat_e[:, None] == jnp.arange(e_num)[None, :]).astype(jnp.int32)
    cs = jnp.cumsum(onehot, axis=0)
    counts = cs[-1]
    pos = jnp.take_along_axis(cs, flat_e[:, None], axis=1)[:, 0] - 1
    nblk_e = (counts + _BLK - 1) // _BLK
    blk_start = jnp.concatenate(
        [jnp.zeros((1,), jnp.int32), jnp.cumsum(nblk_e).astype(jnp.int32)])
    row = (blk_start[:-1] * _BLK)[flat_e] + pos
    rows2 = row.reshape(n, 2)
    r0 = rows2[:, 0].astype(jnp.int32)
    r1 = rows2[:, 1].astype(jnp.int32)
    nvalid = blk_start[-1:].astype(jnp.int32)
    j_idx = jnp.minimum(jnp.arange(_NBLK, dtype=jnp.int32), nvalid[0] - 1)
    block_expert = jnp.searchsorted(
        blk_start[1:], j_idx, side="right").astype(jnp.int32)
    w_sorted = jnp.zeros((_RMAX,), jnp.float32).at[row].set(wnorm.reshape(-1))

    xs = _make_dispatch(n, d)(u, r0, r1)
    ys = _grouped_ffn(block_expert, nvalid, xs, W1, b1, W2, b2, w_sorted)
    out = _make_combine(n, d)(ys, r0, r1)
    return out.reshape(b, s, d)
